# Initial kernel scaffold; baseline (speedup 1.0000x reference)
import jax
import jax.numpy as jnp
from jax.experimental import pallas as pl


def kernel(edge_index, input, W_in0, W_rec0, W_in1, W_rec1):
    raise NotImplementedError("write your pallas kernel here")


# speedup vs baseline: 2.4046x; 2.4046x over previous
import jax
import jax.numpy as jnp
from jax import lax
from jax.experimental import pallas as pl
from jax.experimental.pallas import tpu as pltpu
from jax.experimental.pallas import tpu_sc as plsc

N = 10000
E = 320000
T = 8
F = 128
H = 128
LEAK = 0.9

NC = 2
NS = 16
L = 16

EPT = E // NS
CH = 128
NCHUNK = (EPT + CH - 1) // CH
EPT_PAD = NCHUNK * CH

ACC = 5120
OWN0 = 5000
DUMMY = 5100
RPT = ACC // NS
ZR = 64

_sc_mesh = plsc.VectorSubcoreMesh(core_axis_name="c", subcore_axis_name="s")


def _sc_agg_body(s0_h, s1_h, src_h, dstl_h, out_h,
                 src_v, dstl_v, rows_v, zbuf_v, acc_s, sem0, sem1):
    c = lax.axis_index("c")
    s = lax.axis_index("s")

    pltpu.sync_copy(src_h.at[s], src_v)
    pltpu.sync_copy(dstl_h.at[c, s], dstl_v)

    zk = jnp.zeros((L,), jnp.float32)

    def _zero_row(i, carry):
        for jj in range(H // L):
            zbuf_v[i, pl.ds(jj * L, L)] = zk
        return carry

    lax.fori_loop(0, ZR, _zero_row, 0)

    def zero_acc():
        for k in range(RPT // ZR):
            pltpu.sync_copy(zbuf_v, acc_s.at[pl.ds(s * RPT + k * ZR, ZR)])

    def sweep(state_h):
        pltpu.async_copy(state_h.at[src_v.at[0]], rows_v.at[0], sem0)

        def body(j, carry):
            @pl.when(j % 2 == 1)
            def _():
                pltpu.async_copy(state_h.at[src_v.at[j]], rows_v.at[1], sem1)
                pltpu.make_async_copy(
                    state_h.at[src_v.at[j - 1]], rows_v.at[0], sem0).wait()
                pltpu.sync_copy(rows_v.at[0], acc_s.at[dstl_v.at[j - 1]],
                                add=True)

            @pl.when(j % 2 == 0)
            def _():
                pltpu.async_copy(state_h.at[src_v.at[j]], rows_v.at[0], sem0)
                pltpu.make_async_copy(
                    state_h.at[src_v.at[j - 1]], rows_v.at[1], sem1).wait()
                pltpu.sync_copy(rows_v.at[1], acc_s.at[dstl_v.at[j - 1]],
                                add=True)

            return carry

        lax.fori_loop(1, NCHUNK, body, 0)
        last = NCHUNK - 1
        pltpu.make_async_copy(
            state_h.at[src_v.at[last]], rows_v.at[0], sem0).wait()
        pltpu.sync_copy(rows_v.at[0], acc_s.at[dstl_v.at[last]], add=True)

    for layer, state_h in ((0, s0_h), (1, s1_h)):
        zero_acc()
        plsc.subcore_barrier()
        sweep(state_h)
        plsc.subcore_barrier()
        pltpu.sync_copy(acc_s.at[pl.ds(s * RPT, RPT)],
                        out_h.at[layer, c, pl.ds(s * RPT, RPT)])


_sc_agg = pl.kernel(
    _sc_agg_body,
    out_type=jax.ShapeDtypeStruct((2, NC, ACC, H), jnp.float32),
    mesh=_sc_mesh,
    scratch_types=[
        pltpu.VMEM((NCHUNK, CH), jnp.int32),
        pltpu.VMEM((NCHUNK, CH), jnp.int32),
        pltpu.VMEM((2, CH, H), jnp.float32),
        pltpu.VMEM((ZR, H), jnp.float32),
        pltpu.VMEM_SHARED((ACC, H), jnp.float32),
        pltpu.SemaphoreType.DMA,
        pltpu.SemaphoreType.DMA,
    ],
)

_DN = (((1,), (1,)), ((), ()))
_RB = 1000
_SPLIT = OWN0 // _RB


def _pick(h0, h1):
    i = pl.program_id(0)
    blk = jnp.where(i < _SPLIT, h0[...], h1[...])
    return blk.reshape(_RB, H)


def _tc_step_body(x, a0h0, a0h1, a1h0, a1h1, s0, s1,
                  wi0, wr0, wi1, wr1, o0, o1):
    a0 = _pick(a0h0, a0h1)
    a1 = _pick(a1h0, a1h1)
    u0 = lax.dot_general(x[...], wi0[...], _DN,
                         preferred_element_type=jnp.float32)
    r0 = lax.dot_general(a0, wr0[...], _DN,
                         preferred_element_type=jnp.float32)
    s0n = LEAK * jnp.tanh(u0 + r0) + (1.0 - LEAK) * s0[...]
    u1 = lax.dot_general(s0n, wi1[...], _DN,
                         preferred_element_type=jnp.float32)
    r1 = lax.dot_general(a1, wr1[...], _DN,
                         preferred_element_type=jnp.float32)
    s1n = LEAK * jnp.tanh(u1 + r1) + (1.0 - LEAK) * s1[...]
    o0[...] = s0n
    o1[...] = s1n


def _tc_t0_body(x, wi0, wi1, o0, o1):
    u0 = lax.dot_general(x[...], wi0[...], _DN,
                         preferred_element_type=jnp.float32)
    s0n = LEAK * jnp.tanh(u0)
    u1 = lax.dot_general(s0n, wi1[...], _DN,
                         preferred_element_type=jnp.float32)
    o1[...] = LEAK * jnp.tanh(u1)
    o0[...] = s0n


def _agg_spec(layer, half):
    if half == 0:
        return pl.BlockSpec((1, 1, _RB, H),
                            lambda i: (layer, 0, jnp.minimum(i, _SPLIT - 1), 0))
    return pl.BlockSpec((1, 1, _RB, H),
                        lambda i: (layer, 1, jnp.maximum(i - _SPLIT, 0), 0))


_row_spec = pl.BlockSpec((_RB, H), lambda i: (i, 0))
_w_spec = pl.BlockSpec((H, H), lambda i: (0, 0))
_st_out = (jax.ShapeDtypeStruct((N, H), jnp.float32),
           jax.ShapeDtypeStruct((N, H), jnp.float32))

_tc_step = pl.pallas_call(
    _tc_step_body,
    grid=(N // _RB,),
    in_specs=[_row_spec,
              _agg_spec(0, 0), _agg_spec(0, 1), _agg_spec(1, 0), _agg_spec(1, 1),
              _row_spec, _row_spec,
              _w_spec, _w_spec, _w_spec, _w_spec],
    out_specs=(_row_spec, _row_spec),
    out_shape=_st_out,
)

_tc_t0 = pl.pallas_call(
    _tc_t0_body,
    grid=(N // _RB,),
    in_specs=[_row_spec, _w_spec, _w_spec],
    out_specs=(_row_spec, _row_spec),
    out_shape=_st_out,
)


def kernel(edge_index, input, W_in0, W_rec0, W_in1, W_rec1):
    src = edge_index[0].astype(jnp.int32)
    dst = edge_index[1].astype(jnp.int32)
    d0 = jnp.where(dst < OWN0, dst, DUMMY)
    d1 = jnp.where(dst >= OWN0, dst - OWN0, DUMMY)
    pad = EPT_PAD - EPT
    src_p = jnp.pad(src.reshape(NS, EPT),
                    ((0, 0), (0, pad))).reshape(NS, NCHUNK, CH)
    dstl = jnp.stack([
        jnp.pad(d0.reshape(NS, EPT), ((0, 0), (0, pad)),
                constant_values=DUMMY),
        jnp.pad(d1.reshape(NS, EPT), ((0, 0), (0, pad)),
                constant_values=DUMMY),
    ]).reshape(NC, NS, NCHUNK, CH)

    s0, s1 = _tc_t0(input[0], W_in0, W_in1)
    for t in range(1, T):
        agg = _sc_agg(s0, s1, src_p, dstl)
        s0, s1 = _tc_step(input[t], agg, agg, agg, agg, s0, s1,
                          W_in0, W_rec0, W_in1, W_rec1)
    return s1


# speedup vs baseline: 2.9957x; 1.2458x over previous
import jax
import jax.numpy as jnp
from jax import lax
from jax.experimental import pallas as pl
from jax.experimental.pallas import tpu as pltpu
from jax.experimental.pallas import tpu_sc as plsc

N = 10000
E = 320000
T = 8
F = 128
H = 128
LEAK = 0.9

NC = 2
NS = 16
L = 16

EPT = E // NS
CH = 128
NCHUNK = (EPT + CH - 1) // CH
EPT_PAD = NCHUNK * CH

ACC = 5120
OWN0 = 5000
DUMMY = 5088
RPT = ACC // NS
ZR = 64

_sc_mesh = plsc.VectorSubcoreMesh(core_axis_name="c", subcore_axis_name="s")


NBUF = 4
IDXH = 80


def _sc_agg_body(s0_h, s1_h, src_h, dstl_h, out_h,
                 src_v, dstl_v, rows_v, acc_s,
                 semg0, semg1, semg2, semg3, sems0, sems1, sems2, sems3):
    semg = (semg0, semg1, semg2, semg3)
    sems = (sems0, sems1, sems2, sems3)
    c = lax.axis_index("c")
    s = lax.axis_index("s")

    zk = jnp.zeros((L,), jnp.float32)

    def zero_acc():
        def _zero_row(i, carry):
            for jj in range(H // L):
                rows_v[0, i, pl.ds(jj * L, L)] = zk
            return carry

        lax.fori_loop(0, CH, _zero_row, 0)
        pltpu.sync_copy(rows_v.at[0], acc_s.at[pl.ds(s * RPT, CH)])
        pltpu.sync_copy(rows_v.at[0], acc_s.at[pl.ds(s * RPT + CH, CH)])
        pltpu.sync_copy(rows_v.at[0, pl.ds(0, RPT - 2 * CH)],
                        acc_s.at[pl.ds(s * RPT + 2 * CH, RPT - 2 * CH)])

    def subsweep(state_h, base, m):
        pltpu.sync_copy(src_h.at[s, pl.ds(base, m)], src_v.at[pl.ds(0, m)])
        pltpu.sync_copy(dstl_h.at[c, s, pl.ds(base, m)],
                        dstl_v.at[pl.ds(0, m)])
        def body(j, carry):
            for r in range(NBUF):
                @pl.when(j % NBUF == r)
                def _(r=r):
                    b2 = (r + NBUF - 2) % NBUF

                    @pl.when(j >= NBUF)
                    def _():
                        pltpu.make_async_copy(
                            rows_v.at[r], acc_s.at[dstl_v.at[j - NBUF]],
                            sems[r]).wait()

                    @pl.when(j < m)
                    def _():
                        pltpu.async_copy(state_h.at[src_v.at[j]],
                                         rows_v.at[r], semg[r])

                    @pl.when(j >= 2)
                    def _():
                        pltpu.make_async_copy(
                            state_h.at[src_v.at[j - 2]], rows_v.at[b2],
                            semg[b2]).wait()
                        pltpu.async_copy(rows_v.at[b2],
                                         acc_s.at[dstl_v.at[j - 2]],
                                         sems[b2], add=True)

            return carry

        lax.fori_loop(0, m + 2, body, 0)
        for q in (m - 2, m - 1):
            pltpu.make_async_copy(
                rows_v.at[q % NBUF], acc_s.at[dstl_v.at[q]],
                sems[q % NBUF]).wait()

    for layer, state_h in ((0, s0_h), (1, s1_h)):
        zero_acc()
        plsc.subcore_barrier()
        subsweep(state_h, 0, IDXH)
        subsweep(state_h, IDXH, NCHUNK - IDXH)
        plsc.subcore_barrier()
        pltpu.sync_copy(acc_s.at[pl.ds(s * RPT, RPT)],
                        out_h.at[layer, c, pl.ds(s * RPT, RPT)])


_sc_agg = pl.kernel(
    _sc_agg_body,
    out_type=jax.ShapeDtypeStruct((2, NC, ACC, H), jnp.float32),
    mesh=_sc_mesh,
    scratch_types=[
        pltpu.VMEM((IDXH, CH), jnp.int32),
        pltpu.VMEM((IDXH, CH), jnp.int32),
        pltpu.VMEM((NBUF, CH, H), jnp.float32),
        pltpu.VMEM_SHARED((ACC, H), jnp.float32),
        pltpu.SemaphoreType.DMA,
        pltpu.SemaphoreType.DMA,
        pltpu.SemaphoreType.DMA,
        pltpu.SemaphoreType.DMA,
        pltpu.SemaphoreType.DMA,
        pltpu.SemaphoreType.DMA,
        pltpu.SemaphoreType.DMA,
        pltpu.SemaphoreType.DMA,
    ],
)

_DN = (((1,), (1,)), ((), ()))
_RB = 1000
_SPLIT = OWN0 // _RB


def _pick(h0, h1):
    i = pl.program_id(0)
    blk = jnp.where(i < _SPLIT, h0[...], h1[...])
    return blk.reshape(_RB, H)


def _tc_step_body(x, a0h0, a0h1, a1h0, a1h1, s0, s1,
                  wi0, wr0, wi1, wr1, o0, o1):
    a0 = _pick(a0h0, a0h1)
    a1 = _pick(a1h0, a1h1)
    u0 = lax.dot_general(x[...], wi0[...], _DN,
                         preferred_element_type=jnp.float32)
    r0 = lax.dot_general(a0, wr0[...], _DN,
                         preferred_element_type=jnp.float32)
    s0n = LEAK * jnp.tanh(u0 + r0) + (1.0 - LEAK) * s0[...]
    u1 = lax.dot_general(s0n, wi1[...], _DN,
                         preferred_element_type=jnp.float32)
    r1 = lax.dot_general(a1, wr1[...], _DN,
                         preferred_element_type=jnp.float32)
    s1n = LEAK * jnp.tanh(u1 + r1) + (1.0 - LEAK) * s1[...]
    o0[...] = s0n
    o1[...] = s1n


def _tc_t0_body(x, wi0, wi1, o0, o1):
    u0 = lax.dot_general(x[...], wi0[...], _DN,
                         preferred_element_type=jnp.float32)
    s0n = LEAK * jnp.tanh(u0)
    u1 = lax.dot_general(s0n, wi1[...], _DN,
                         preferred_element_type=jnp.float32)
    o1[...] = LEAK * jnp.tanh(u1)
    o0[...] = s0n


def _agg_spec(layer, half):
    if half == 0:
        return pl.BlockSpec((1, 1, _RB, H),
                            lambda i: (layer, 0, jnp.minimum(i, _SPLIT - 1), 0))
    return pl.BlockSpec((1, 1, _RB, H),
                        lambda i: (layer, 1, jnp.maximum(i - _SPLIT, 0), 0))


_row_spec = pl.BlockSpec((_RB, H), lambda i: (i, 0))
_w_spec = pl.BlockSpec((H, H), lambda i: (0, 0))
_st_out = (jax.ShapeDtypeStruct((N, H), jnp.float32),
           jax.ShapeDtypeStruct((N, H), jnp.float32))

_tc_step = pl.pallas_call(
    _tc_step_body,
    grid=(N // _RB,),
    in_specs=[_row_spec,
              _agg_spec(0, 0), _agg_spec(0, 1), _agg_spec(1, 0), _agg_spec(1, 1),
              _row_spec, _row_spec,
              _w_spec, _w_spec, _w_spec, _w_spec],
    out_specs=(_row_spec, _row_spec),
    out_shape=_st_out,
)

_tc_t0 = pl.pallas_call(
    _tc_t0_body,
    grid=(N // _RB,),
    in_specs=[_row_spec, _w_spec, _w_spec],
    out_specs=(_row_spec, _row_spec),
    out_shape=_st_out,
)


def kernel(edge_index, input, W_in0, W_rec0, W_in1, W_rec1):
    src = edge_index[0].astype(jnp.int32)
    dst = edge_index[1].astype(jnp.int32)
    spread = jnp.arange(E, dtype=jnp.int32) % (ACC - DUMMY)
    d0 = jnp.where(dst < OWN0, dst, DUMMY + spread)
    d1 = jnp.where(dst >= OWN0, dst - OWN0, DUMMY + spread)
    pad = EPT_PAD - EPT
    src_p = jnp.pad(src.reshape(NS, EPT),
                    ((0, 0), (0, pad))).reshape(NS, NCHUNK, CH)
    dstl = jnp.stack([
        jnp.pad(d0.reshape(NS, EPT), ((0, 0), (0, pad)),
                constant_values=DUMMY),
        jnp.pad(d1.reshape(NS, EPT), ((0, 0), (0, pad)),
                constant_values=DUMMY),
    ]).reshape(NC, NS, NCHUNK, CH)

    s0, s1 = _tc_t0(input[0], W_in0, W_in1)
    for t in range(1, T):
        agg = _sc_agg(s0, s1, src_p, dstl)
        s0, s1 = _tc_step(input[t], agg, agg, agg, agg, s0, s1,
                          W_in0, W_rec0, W_in1, W_rec1)
    return s1
